# Initial kernel scaffold; baseline (speedup 1.0000x reference)
import jax
import jax.numpy as jnp
from jax.experimental import pallas as pl


def kernel(query, support, query_left_connections, query_left_degrees, query_right_connections, query_right_degrees, support_left_connections, support_left_degrees, support_right_connections, support_right_degrees, symbol_emb, gcn_w_weight, gcn_w_bias, se_proj1_w, se_proj1_b, se_proj2_w, se_proj2_b, se_ln_a, se_ln_b, lstm_w_ih, lstm_w_hh, lstm_b_ih, lstm_b_hh):
    raise NotImplementedError("write your pallas kernel here")


# speedup vs baseline: 2.8746x; 2.8746x over previous
import functools

import jax
import jax.numpy as jnp
from jax import lax
from jax.experimental import pallas as pl
from jax.experimental.pallas import tpu as pltpu
from jax.experimental.pallas import tpu_sc as plsc

EMBED = 128
NBR = 200
BATCH = 1024
FEW = 5
D_MODEL = 256
HID = 512
STEPS = 4

NW = 32
ROWS_PAD = 1056
RPW = ROWS_PAD // NW
NSTREAM = 4
NJ = RPW * NSTREAM
NBUF = 3
C0, C1 = 96, 104


IDX_PW = NJ * NBR
OUT_PW = RPW * 4 * EMBED


def _sc_gather_sums(table, idx_flat):
    mesh = plsc.VectorSubcoreMesh(core_axis_name="c", subcore_axis_name="s")

    @functools.partial(
        pl.kernel,
        mesh=mesh,
        out_type=jax.ShapeDtypeStruct((ROWS_PAD * 4 * EMBED,), jnp.float32),
        scratch_types=[
            pltpu.VMEM((IDX_PW,), jnp.int32),
            pltpu.VMEM((NBR, EMBED), jnp.float32),
            pltpu.VMEM((NBR, EMBED), jnp.float32),
            pltpu.VMEM((NBR, EMBED), jnp.float32),
            pltpu.VMEM((OUT_PW,), jnp.float32),
            pltpu.SemaphoreType.DMA,
            pltpu.SemaphoreType.DMA,
            pltpu.SemaphoreType.DMA,
        ],
    )
    def sc_fn(table_hbm, idx_hbm, out_hbm, idx_v, b0, b1, b2, ostage, s0, s1, s2):
        wid = lax.axis_index("s") * 2 + lax.axis_index("c")
        pltpu.sync_copy(
            idx_hbm.at[pl.ds(pl.multiple_of(wid * IDX_PW, 8), IDX_PW)], idx_v)

        bufs = (b0, b1, b2)
        sems = (s0, s1, s2)

        def issue(j, buf, sem):
            off = pl.multiple_of(j * NBR, 8)
            pltpu.async_copy(
                table_hbm.at[idx_v.at[pl.ds(off, C0)]],
                buf.at[pl.ds(0, C0)], sem)
            off2 = pl.multiple_of(j * NBR + C0, 8)
            pltpu.async_copy(
                table_hbm.at[idx_v.at[pl.ds(off2, C1)]],
                buf.at[pl.ds(C0, C1)], sem)

        def wait(buf, sem):
            pltpu.make_async_copy(
                table_hbm.at[idx_v.at[pl.ds(0, C0)]],
                buf.at[pl.ds(0, C0)], sem).wait()
            pltpu.make_async_copy(
                table_hbm.at[idx_v.at[pl.ds(C0, C1)]],
                buf.at[pl.ds(C0, C1)], sem).wait()

        def accumulate(buf):
            def rbody(r, accs):
                a = list(accs)
                for u in range(4):
                    row = 4 * r + u
                    for c in range(EMBED // 16):
                        a[c] = a[c] + buf[row, pl.ds(c * 16, 16)]
                return tuple(a)
            zero = tuple(jnp.zeros((16,), jnp.float32) for _ in range(EMBED // 16))
            return lax.fori_loop(0, NBR // 4, rbody, zero)

        for t in range(NBUF):
            issue(jnp.int32(t), bufs[t], sems[t])

        def gbody(g, carry):
            for b in range(NBUF):
                j = NBUF * g + b
                wait(bufs[b], sems[b])
                accs = accumulate(bufs[b])
                for c in range(EMBED // 16):
                    off = pl.multiple_of(j * EMBED + c * 16, 16)
                    ostage[pl.ds(off, 16)] = accs[c]

                @pl.when(j + NBUF < NJ)
                def _():
                    issue(j + NBUF, bufs[b], sems[b])
            return carry

        lax.fori_loop(0, NJ // NBUF, gbody, jnp.int32(0))
        pltpu.sync_copy(
            ostage, out_hbm.at[pl.ds(pl.multiple_of(wid * OUT_PW, 8), OUT_PW)])

    return sc_fn(table, idx_flat)


def _tc_dense(sums_q, qdl, qdr, sums_s, sdl, sdr,
              gcn_w, gcn_b, p1w, p1b, p2w, p2b, ln_a, ln_b,
              wih, whh1, whh2, lstm_b):
    BLK = 128
    grid = BATCH // BLK

    def body(sq_ref, qdl_ref, qdr_ref, ss_ref, sdl_ref, sdr_ref,
             gw_ref, gb_ref, p1w_ref, p1b_ref, p2w_ref, p2b_ref,
             la_ref, lb_ref, wih_ref, whh1_ref, whh2_ref, bias_ref,
             out_ref):
        gw = gw_ref[...]
        gb = gb_ref[...]

        def nbr_enc(sums, dl, dr):
            left = jnp.tanh(
                (jax.lax.dot_general(sums[:, :256], gw, (((1,), (1,)), ((), ())),
                                     preferred_element_type=jnp.float32)
                 + NBR * gb) / dl)
            right = jnp.tanh(
                (jax.lax.dot_general(sums[:, 256:], gw, (((1,), (1,)), ((), ())),
                                     preferred_element_type=jnp.float32)
                 + NBR * gb) / dr)
            return jnp.concatenate([left, right], axis=-1)

        p1w = p1w_ref[...]
        p2w = p2w_ref[...]

        def se(x):
            h = jax.lax.dot_general(x, p1w, (((1,), (1,)), ((), ())),
                                    preferred_element_type=jnp.float32)
            h = jnp.maximum(h + p1b_ref[...], 0.0)
            h = jax.lax.dot_general(h, p2w, (((1,), (1,)), ((), ())),
                                    preferred_element_type=jnp.float32)
            h = h + p2b_ref[...] + x
            mu = jnp.mean(h, axis=-1, keepdims=True)
            var = jnp.sum((h - mu) ** 2, axis=-1, keepdims=True) / (D_MODEL - 1)
            return (h - mu) / (jnp.sqrt(var) + 1e-3) * la_ref[...] + lb_ref[...]

        s_g = se(nbr_enc(ss_ref[...], sdl_ref[...], sdr_ref[...]))
        row = lax.broadcasted_iota(jnp.int32, (8, 1), 0)
        s_mean = jnp.sum(jnp.where(row < FEW, s_g, 0.0), axis=0,
                         keepdims=True) / FEW

        q_g = se(nbr_enc(sq_ref[...], qdl_ref[...], qdr_ref[...]))

        qWih = jax.lax.dot_general(q_g, wih_ref[...], (((1,), (1,)), ((), ())),
                                   preferred_element_type=jnp.float32)
        sW2 = jax.lax.dot_general(s_mean, whh2_ref[...], (((1,), (1,)), ((), ())),
                                  preferred_element_type=jnp.float32)
        bias = bias_ref[...]
        whh1 = whh1_ref[...]

        c = jnp.zeros((BLK, HID), jnp.float32)
        h = q_g
        for step in range(STEPS):
            if step == 0:
                gates = qWih + bias
            else:
                gates = qWih + bias + sW2 + jax.lax.dot_general(
                    h, whh1, (((1,), (1,)), ((), ())),
                    preferred_element_type=jnp.float32)
            ii = jax.nn.sigmoid(gates[:, 0 * HID:1 * HID])
            ff = jax.nn.sigmoid(gates[:, 1 * HID:2 * HID])
            gg = jnp.tanh(gates[:, 2 * HID:3 * HID])
            oo = jax.nn.sigmoid(gates[:, 3 * HID:4 * HID])
            c = ff * c + ii * gg
            h = q_g + (oo * jnp.tanh(c))[:, :D_MODEL]
        out_ref[...] = jnp.sum(h * s_mean, axis=-1)[None, None, :]

    full = lambda shape: pl.BlockSpec(shape, lambda i: (0,) * len(shape))
    return pl.pallas_call(
        body,
        grid=(grid,),
        in_specs=[
            pl.BlockSpec((BLK, 512), lambda i: (i, 0)),
            pl.BlockSpec((BLK, 1), lambda i: (i, 0)),
            pl.BlockSpec((BLK, 1), lambda i: (i, 0)),
            full((8, 512)), full((8, 1)), full((8, 1)),
            full((EMBED, 256)), full((1, EMBED)),
            full((512, 256)), full((1, 512)),
            full((256, 512)), full((1, 256)),
            full((1, 256)), full((1, 256)),
            full((4 * HID, 256)), full((4 * HID, 256)), full((4 * HID, 256)),
            full((1, 4 * HID)),
        ],
        out_specs=pl.BlockSpec((1, 1, BLK), lambda i: (i, 0, 0)),
        out_shape=jax.ShapeDtypeStruct((grid, 1, BLK), jnp.float32),
        compiler_params=pltpu.CompilerParams(
            dimension_semantics=("arbitrary",)),
    )(sums_q, qdl, qdr, sums_s, sdl, sdr,
      gcn_w, gcn_b, p1w, p1b, p2w, p2b, ln_a, ln_b,
      wih, whh1, whh2, lstm_b)


def kernel(query, support, query_left_connections, query_left_degrees,
           query_right_connections, query_right_degrees,
           support_left_connections, support_left_degrees,
           support_right_connections, support_right_degrees,
           symbol_emb, gcn_w_weight, gcn_w_bias,
           se_proj1_w, se_proj1_b, se_proj2_w, se_proj2_b, se_ln_a, se_ln_b,
           lstm_w_ih, lstm_w_hh, lstm_b_ih, lstm_b_hh):
    i32 = jnp.int32
    qlc = query_left_connections.astype(i32)
    qrc = query_right_connections.astype(i32)
    slc = support_left_connections.astype(i32)
    src = support_right_connections.astype(i32)

    idx_q = jnp.stack([qlc[:, :, 0], qlc[:, :, 1],
                       qrc[:, :, 0], qrc[:, :, 1]], axis=1)
    idx_s = jnp.stack([slc[:, :, 0], slc[:, :, 1],
                       src[:, :, 0], src[:, :, 1]], axis=1)
    pad = jnp.zeros((ROWS_PAD - BATCH - FEW, NSTREAM, NBR), i32)
    idx = jnp.concatenate([idx_q, idx_s, pad], axis=0)

    sums = _sc_gather_sums(symbol_emb.astype(jnp.float32), idx.reshape(-1))
    sums = sums.reshape(ROWS_PAD, 4 * EMBED)

    sums_q = sums[:BATCH]
    sums_s = jnp.concatenate(
        [sums[BATCH:BATCH + FEW], jnp.zeros((3, 4 * EMBED), jnp.float32)], axis=0)
    qdl = query_left_degrees.astype(jnp.float32)[:, None]
    qdr = query_right_degrees.astype(jnp.float32)[:, None]
    sdl = jnp.concatenate([support_left_degrees.astype(jnp.float32),
                           jnp.ones((3,), jnp.float32)])[:, None]
    sdr = jnp.concatenate([support_right_degrees.astype(jnp.float32),
                           jnp.ones((3,), jnp.float32)])[:, None]

    scores = _tc_dense(
        sums_q, qdl, qdr, sums_s, sdl, sdr,
        gcn_w_weight, gcn_w_bias[None, :],
        se_proj1_w, se_proj1_b[None, :], se_proj2_w, se_proj2_b[None, :],
        se_ln_a[None, :], se_ln_b[None, :],
        lstm_w_ih, lstm_w_hh[:, :D_MODEL], lstm_w_hh[:, D_MODEL:],
        (lstm_b_ih + lstm_b_hh)[None, :])
    return scores.reshape(-1)
